# baseline (speedup 1.0000x reference)
import jax
import jax.numpy as jnp
from jax import lax
from jax.experimental import pallas as pl
from jax.experimental.pallas import tpu as pltpu
from jax.experimental.pallas import tpu_sc as plsc

VOCAB = 1000000
EMBED_DIM = 64
BATCH = 16384
HIST = 50
NUM_IDX = BATCH * HIST

NUM_WORKERS = 32
B_PER_W = NUM_IDX // NUM_WORKERS
ROWS_PER_W = BATCH // NUM_WORKERS
CHUNKB = 4
CHUNK = CHUNKB * HIST
GATHER_SPLITS = ((0, 128), (128, 72))
NBUF = 4
N_CHUNKS = ROWS_PER_W // CHUNKB
N_GROUPS = N_CHUNKS // NBUF


HALF_VOCAB = VOCAB // 2
TW = 3200
TGRID = (VOCAB + 2 * TW - 1) // (2 * TW)
EDGE_START = (TGRID - 1) * 2 * TW
EDGE_HALF = (VOCAB - EDGE_START) // 2


def _tc_pack(t_transposed):
    def body(x_ref, o_ref):
        i = pl.program_id(0)

        @pl.when(i < TGRID - 1)
        def _():
            o_ref[:, 0:EMBED_DIM] = x_ref[:, 0:TW].T
            o_ref[:, EMBED_DIM:] = x_ref[:, TW:].T

        @pl.when(i == TGRID - 1)
        def _():
            o_ref[0:EDGE_HALF, 0:EMBED_DIM] = x_ref[:, 0:EDGE_HALF].T
            o_ref[0:EDGE_HALF, EMBED_DIM:] = x_ref[
                :, EDGE_HALF : 2 * EDGE_HALF
            ].T

    return pl.pallas_call(
        body,
        grid=(TGRID,),
        in_specs=[pl.BlockSpec((EMBED_DIM, 2 * TW), lambda i: (0, i))],
        out_specs=pl.BlockSpec((TW, 2 * EMBED_DIM), lambda i: (i, 0)),
        out_shape=jax.ShapeDtypeStruct((HALF_VOCAB, 2 * EMBED_DIM), jnp.float32),
        compiler_params=pltpu.CompilerParams(
            dimension_semantics=("parallel",),
        ),
    )(t_transposed)


def _sc_gather(x_flat, table):
    mesh = plsc.VectorSubcoreMesh(core_axis_name="c", subcore_axis_name="s")

    @pl.kernel(
        out_type=jax.ShapeDtypeStruct((BATCH, HIST, EMBED_DIM), jnp.float32),
        mesh=mesh,
        compiler_params=pltpu.CompilerParams(use_tc_tiling_on_sc=False),
        scratch_types=(
            [
                pltpu.VMEM((B_PER_W,), jnp.int32),
                pltpu.VMEM((NBUF, CHUNK, EMBED_DIM), jnp.float32),
            ]
            + [pltpu.SemaphoreType.DMA] * (2 * NBUF)
        ),
    )
    def gather_kernel(table_hbm, idx_hbm, out_hbm, idx_all, rows_v, *sems):
        gsem = sems[:NBUF]
        wsem = sems[NBUF:]
        wid = lax.axis_index("s") * 2 + lax.axis_index("c")
        base = wid * B_PER_W
        rbase = wid * ROWS_PER_W
        pltpu.sync_copy(idx_hbm.at[pl.ds(base, B_PER_W)], idx_all)

        def enq_gather(c, b):
            for off, n in GATHER_SPLITS:
                pltpu.async_copy(
                    table_hbm.at[idx_all.at[pl.ds(c * CHUNK + off, n)]],
                    rows_v.at[b, pl.ds(off, n)],
                    gsem[b],
                )

        def wait_gather(c, b):
            for off, n in GATHER_SPLITS:
                pltpu.make_async_copy(
                    table_hbm.at[idx_all.at[pl.ds(c * CHUNK + off, n)]],
                    rows_v.at[b, pl.ds(off, n)],
                    gsem[b],
                ).wait()

        def enq_write(c, b):
            for j in range(CHUNKB):
                pltpu.async_copy(
                    rows_v.at[b, pl.ds(j * HIST, HIST)],
                    out_hbm.at[rbase + c * CHUNKB + j],
                    wsem[b],
                )

        def wait_write(c, b):
            for j in range(CHUNKB):
                pltpu.make_async_copy(
                    rows_v.at[b, pl.ds(j * HIST, HIST)],
                    out_hbm.at[rbase + c * CHUNKB + j],
                    wsem[b],
                ).wait()

        for b in range(NBUF):
            enq_gather(b, b)
        for b in range(NBUF):
            wait_gather(b, b)
            enq_write(b, b)

        @pl.loop(1, N_GROUPS)
        def _(g):
            c0 = g * NBUF
            for b in range(NBUF):
                wait_write(c0 - NBUF + b, b)
                enq_gather(c0 + b, b)
            for b in range(NBUF):
                wait_gather(c0 + b, b)
                enq_write(c0 + b, b)

        for b in range(NBUF):
            wait_write(N_CHUNKS - NBUF + b, b)

    return gather_kernel(table, x_flat)


@jax.jit
def kernel(x, table):
    x_flat = x.reshape(NUM_IDX).astype(jnp.int32)
    blk = x_flat // (2 * TW)
    j = x_flat - blk * (2 * TW)
    half = j // TW
    x_lin = blk * (2 * TW) + 2 * (j - half * TW) + half
    je = x_flat - EDGE_START
    he = je // EDGE_HALF
    x_lin_edge = EDGE_START + 2 * (je - he * EDGE_HALF) + he
    x_lin = jnp.where(x_flat < EDGE_START, x_lin, x_lin_edge)
    table_lin = _tc_pack(table.T).reshape(VOCAB, EMBED_DIM)
    return _sc_gather(x_lin, table_lin)


# speedup vs baseline: 1.1100x; 1.1100x over previous
import jax
import jax.numpy as jnp
from jax import lax
from jax.experimental import pallas as pl
from jax.experimental.pallas import tpu as pltpu
from jax.experimental.pallas import tpu_sc as plsc

VOCAB = 1000000
EMBED_DIM = 64
BATCH = 16384
HIST = 50
NUM_IDX = BATCH * HIST

NUM_WORKERS = 32
B_PER_W = NUM_IDX // NUM_WORKERS

HALF_VOCAB = VOCAB // 2
TW = 3200
TGRID = (VOCAB + 2 * TW - 1) // (2 * TW)
EDGE_START = (TGRID - 1) * 2 * TW
EDGE_HALF = (VOCAB - EDGE_START) // 2

CHUNK = 256
WINDOWS_PER_H = BATCH // CHUNK
NBUF = 4
N_CHUNKS = B_PER_W // CHUNK
N_GROUPS = N_CHUNKS // NBUF
GATHER_W = 128


def _tc_pack(t_transposed):
    def body(x_ref, o_ref):
        i = pl.program_id(0)

        @pl.when(i < TGRID - 1)
        def _():
            o_ref[:, 0:EMBED_DIM] = x_ref[:, 0:TW].T
            o_ref[:, EMBED_DIM:] = x_ref[:, TW:].T

        @pl.when(i == TGRID - 1)
        def _():
            o_ref[0:EDGE_HALF, 0:EMBED_DIM] = x_ref[:, 0:EDGE_HALF].T
            o_ref[0:EDGE_HALF, EMBED_DIM:] = x_ref[
                :, EDGE_HALF : 2 * EDGE_HALF
            ].T

    return pl.pallas_call(
        body,
        grid=(TGRID,),
        in_specs=[pl.BlockSpec((EMBED_DIM, 2 * TW), lambda i: (0, i))],
        out_specs=pl.BlockSpec((TW, 2 * EMBED_DIM), lambda i: (i, 0)),
        out_shape=jax.ShapeDtypeStruct((HALF_VOCAB, 2 * EMBED_DIM), jnp.float32),
        compiler_params=pltpu.CompilerParams(
            dimension_semantics=("parallel",),
        ),
    )(t_transposed)


def _sc_gather(x_flat, table):
    mesh = plsc.VectorSubcoreMesh(core_axis_name="c", subcore_axis_name="s")

    @pl.kernel(
        out_type=jax.ShapeDtypeStruct((HIST, BATCH, EMBED_DIM), jnp.float32),
        mesh=mesh,
        compiler_params=pltpu.CompilerParams(use_tc_tiling_on_sc=False),
        scratch_types=(
            [
                pltpu.VMEM((B_PER_W,), jnp.int32),
                pltpu.VMEM((NBUF, CHUNK, EMBED_DIM), jnp.float32),
            ]
            + [pltpu.SemaphoreType.DMA] * (2 * NBUF)
        ),
    )
    def gather_kernel(table_hbm, idx_hbm, out_hbm, idx_all, rows_v, *sems):
        gsem = sems[:NBUF]
        wsem = sems[NBUF:]
        wid = lax.axis_index("s") * 2 + lax.axis_index("c")
        base = wid * B_PER_W
        tbase = wid * N_CHUNKS
        pltpu.sync_copy(idx_hbm.at[pl.ds(base, B_PER_W)], idx_all)

        def enq_gather(c, b):
            for j in range(CHUNK // GATHER_W):
                pltpu.async_copy(
                    table_hbm.at[
                        idx_all.at[pl.ds(c * CHUNK + j * GATHER_W, GATHER_W)]
                    ],
                    rows_v.at[b, pl.ds(j * GATHER_W, GATHER_W)],
                    gsem[b],
                )

        def wait_gather(c, b):
            for j in range(CHUNK // GATHER_W):
                pltpu.make_async_copy(
                    table_hbm.at[
                        idx_all.at[pl.ds(c * CHUNK + j * GATHER_W, GATHER_W)]
                    ],
                    rows_v.at[b, pl.ds(j * GATHER_W, GATHER_W)],
                    gsem[b],
                ).wait()

        def _dst(c):
            t = tbase + c
            h = t // WINDOWS_PER_H
            b0 = (t - h * WINDOWS_PER_H) * CHUNK
            return out_hbm.at[h, pl.ds(b0, CHUNK)]

        def enq_write(c, b):
            pltpu.async_copy(rows_v.at[b], _dst(c), wsem[b])

        def wait_write(c, b):
            pltpu.make_async_copy(rows_v.at[b], _dst(c), wsem[b]).wait()

        for b in range(NBUF):
            enq_gather(b, b)
        for b in range(NBUF):
            wait_gather(b, b)
            enq_write(b, b)

        @pl.loop(1, N_GROUPS)
        def _(g):
            c0 = g * NBUF
            for b in range(NBUF):
                wait_write(c0 - NBUF + b, b)
                enq_gather(c0 + b, b)
            for b in range(NBUF):
                wait_gather(c0 + b, b)
                enq_write(c0 + b, b)

        for b in range(NBUF):
            wait_write(N_CHUNKS - NBUF + b, b)

    return gather_kernel(table, x_flat)


HB = BATCH // 2


def _tc_merge(flat128):
    def body(x_ref, o_ref):
        o_ref[0, :, 0:HB] = x_ref[:, 0:EMBED_DIM].T
        o_ref[0, :, HB:] = x_ref[:, EMBED_DIM:].T

    return pl.pallas_call(
        body,
        grid=(HIST,),
        in_specs=[pl.BlockSpec((HB, 2 * EMBED_DIM), lambda h: (h, 0))],
        out_specs=pl.BlockSpec((1, EMBED_DIM, BATCH), lambda h: (h, 0, 0)),
        out_shape=jax.ShapeDtypeStruct((HIST, EMBED_DIM, BATCH), jnp.float32),
        compiler_params=pltpu.CompilerParams(
            dimension_semantics=("parallel",),
        ),
    )(flat128)


@jax.jit
def kernel(x, table):
    xt = x.astype(jnp.int32).T
    x_perm = jnp.stack([xt[:, :HB], xt[:, HB:]], axis=-1)
    x_flat = x_perm.reshape(NUM_IDX)
    blk = x_flat // (2 * TW)
    j = x_flat - blk * (2 * TW)
    half = j // TW
    x_lin = blk * (2 * TW) + 2 * (j - half * TW) + half
    je = x_flat - EDGE_START
    he = je // EDGE_HALF
    x_lin_edge = EDGE_START + 2 * (je - he * EDGE_HALF) + he
    x_lin = jnp.where(x_flat < EDGE_START, x_lin, x_lin_edge)

    table_lin = _tc_pack(table.T).reshape(VOCAB, EMBED_DIM)
    hmajor = _sc_gather(x_lin, table_lin)
    out = _tc_merge(hmajor.reshape(NUM_IDX // 2, 2 * EMBED_DIM))
    return out.transpose(2, 0, 1)


# speedup vs baseline: 1.5328x; 1.3810x over previous
import jax
import jax.numpy as jnp
from jax import lax
from jax.experimental import pallas as pl
from jax.experimental.pallas import tpu as pltpu
from jax.experimental.pallas import tpu_sc as plsc

VOCAB = 1000000
EMBED_DIM = 64
BATCH = 16384
HIST = 50
NUM_IDX = BATCH * HIST

NUM_WORKERS = 32
B_PER_W = NUM_IDX // NUM_WORKERS

HALF_VOCAB = VOCAB // 2
TW = 3200
TGRID = (VOCAB + 2 * TW - 1) // (2 * TW)
EDGE_START = (TGRID - 1) * 2 * TW
EDGE_HALF = (VOCAB - EDGE_START) // 2

CHUNK = 256
WINDOWS_PER_H = BATCH // CHUNK
NBUF = 4
N_CHUNKS = B_PER_W // CHUNK
N_GROUPS = N_CHUNKS // NBUF
GATHER_W = 128


def _tc_pack(t_transposed):
    def body(x_ref, o_ref):
        i = pl.program_id(0)

        @pl.when(i < TGRID - 1)
        def _():
            o_ref[:, 0:EMBED_DIM] = x_ref[:, 0:TW].T
            o_ref[:, EMBED_DIM:] = x_ref[:, TW:].T

        @pl.when(i == TGRID - 1)
        def _():
            o_ref[0:EDGE_HALF, 0:EMBED_DIM] = x_ref[:, 0:EDGE_HALF].T
            o_ref[0:EDGE_HALF, EMBED_DIM:] = x_ref[
                :, EDGE_HALF : 2 * EDGE_HALF
            ].T

    return pl.pallas_call(
        body,
        grid=(TGRID,),
        in_specs=[pl.BlockSpec((EMBED_DIM, 2 * TW), lambda i: (0, i))],
        out_specs=pl.BlockSpec((TW, 2 * EMBED_DIM), lambda i: (i, 0)),
        out_shape=jax.ShapeDtypeStruct((HALF_VOCAB, 2 * EMBED_DIM), jnp.float32),
        compiler_params=pltpu.CompilerParams(
            dimension_semantics=("parallel",),
        ),
    )(t_transposed)


def _sc_gather(x_flat, table):
    mesh = plsc.VectorSubcoreMesh(core_axis_name="c", subcore_axis_name="s")

    @pl.kernel(
        out_type=jax.ShapeDtypeStruct((HIST, BATCH, EMBED_DIM), jnp.float32),
        mesh=mesh,
        compiler_params=pltpu.CompilerParams(use_tc_tiling_on_sc=False),
        scratch_types=(
            [
                pltpu.VMEM((B_PER_W,), jnp.int32),
                pltpu.VMEM((NBUF, CHUNK, EMBED_DIM), jnp.float32),
            ]
            + [pltpu.SemaphoreType.DMA] * (2 * NBUF)
        ),
    )
    def gather_kernel(table_hbm, idx_hbm, out_hbm, idx_all, rows_v, *sems):
        gsem = sems[:NBUF]
        wsem = sems[NBUF:]
        wid = lax.axis_index("s") * 2 + lax.axis_index("c")
        base = wid * B_PER_W
        tbase = wid * N_CHUNKS
        pltpu.sync_copy(idx_hbm.at[pl.ds(base, B_PER_W)], idx_all)

        def enq_gather(c, b):
            for j in range(CHUNK // GATHER_W):
                pltpu.async_copy(
                    table_hbm.at[
                        idx_all.at[pl.ds(c * CHUNK + j * GATHER_W, GATHER_W)]
                    ],
                    rows_v.at[b, pl.ds(j * GATHER_W, GATHER_W)],
                    gsem[b],
                )

        def wait_gather(c, b):
            for j in range(CHUNK // GATHER_W):
                pltpu.make_async_copy(
                    table_hbm.at[
                        idx_all.at[pl.ds(c * CHUNK + j * GATHER_W, GATHER_W)]
                    ],
                    rows_v.at[b, pl.ds(j * GATHER_W, GATHER_W)],
                    gsem[b],
                ).wait()

        def _dst(c):
            t = tbase + c
            h = t // WINDOWS_PER_H
            b0 = (t - h * WINDOWS_PER_H) * CHUNK
            return out_hbm.at[h, pl.ds(b0, CHUNK)]

        def enq_write(c, b):
            pltpu.async_copy(rows_v.at[b], _dst(c), wsem[b])

        def wait_write(c, b):
            pltpu.make_async_copy(rows_v.at[b], _dst(c), wsem[b]).wait()

        for b in range(NBUF):
            enq_gather(b, b)
        for b in range(NBUF):
            wait_gather(b, b)
            enq_write(b, b)

        @pl.loop(1, N_GROUPS)
        def _(g):
            c0 = g * NBUF
            for b in range(NBUF):
                wait_write(c0 - NBUF + b, b)
                enq_gather(c0 + b, b)
            for b in range(NBUF):
                wait_gather(c0 + b, b)
                enq_write(c0 + b, b)

        for b in range(NBUF):
            wait_write(N_CHUNKS - NBUF + b, b)

    return gather_kernel(table, x_flat)


HB = BATCH // 2
IDX_ROWS = NUM_IDX // 128
RPP = BATCH // 128


def _sc_idx_interleave(x128):
    mesh = plsc.VectorSubcoreMesh(core_axis_name="c", subcore_axis_name="s")

    @pl.kernel(
        out_type=jax.ShapeDtypeStruct((IDX_ROWS, 128), jnp.int32),
        mesh=mesh,
        compiler_params=pltpu.CompilerParams(needs_layout_passes=False),
        scratch_types=[
            pltpu.VMEM((RPP, 128), jnp.int32),
            pltpu.VMEM((RPP, 128), jnp.int32),
        ],
    )
    def k(x_hbm, o_hbm, plane_v, out_v):
        wid = lax.axis_index("s") * 2 + lax.axis_index("c")
        iota = lax.iota(jnp.int32, 16)
        c0 = (iota // 2) + (iota % 2) * HB

        def do_plane(h):
            pltpu.sync_copy(x_hbm.at[pl.ds(h * RPP, RPP)], plane_v)

            @pl.loop(0, RPP)
            def _(r):
                for q in range(8):
                    cols = c0 + (r * 64 + 8 * q)
                    vals = plsc.load_gather(
                        plane_v, [cols >> 7, cols & 127]
                    )
                    out_v[r, pl.ds(16 * q, 16)] = vals

            pltpu.sync_copy(out_v, o_hbm.at[pl.ds(h * RPP, RPP)])

        do_plane(wid)

        @pl.when(wid + 32 < HIST)
        def _():
            do_plane(wid + 32)

    return k(x128)


def _tc_merge(flat128):
    def body(x_ref, o_ref):
        o_ref[0, :, 0:HB] = x_ref[:, 0:EMBED_DIM].T
        o_ref[0, :, HB:] = x_ref[:, EMBED_DIM:].T

    return pl.pallas_call(
        body,
        grid=(HIST,),
        in_specs=[pl.BlockSpec((HB, 2 * EMBED_DIM), lambda h: (h, 0))],
        out_specs=pl.BlockSpec((1, EMBED_DIM, BATCH), lambda h: (h, 0, 0)),
        out_shape=jax.ShapeDtypeStruct((HIST, EMBED_DIM, BATCH), jnp.float32),
        compiler_params=pltpu.CompilerParams(
            dimension_semantics=("parallel",),
        ),
    )(flat128)


@jax.jit
def kernel(x, table):
    xi = x.astype(jnp.int32)
    blk = xi // (2 * TW)
    j = xi - blk * (2 * TW)
    half = j // TW
    xm = blk * (2 * TW) + 2 * (j - half * TW) + half
    je = xi - EDGE_START
    he = je // EDGE_HALF
    xm_edge = EDGE_START + 2 * (je - he * EDGE_HALF) + he
    xm = jnp.where(xi < EDGE_START, xm, xm_edge)

    x128 = xm.T.reshape(IDX_ROWS, 128)
    x_lin = _sc_idx_interleave(x128).reshape(NUM_IDX)

    table_lin = _tc_pack(table.T).reshape(VOCAB, EMBED_DIM)
    hmajor = _sc_gather(x_lin, table_lin)
    out = _tc_merge(hmajor.reshape(NUM_IDX // 2, 2 * EMBED_DIM))
    return out.transpose(2, 0, 1)


# speedup vs baseline: 1.6317x; 1.0645x over previous
import jax
import jax.numpy as jnp
from jax import lax
from jax.experimental import pallas as pl
from jax.experimental.pallas import tpu as pltpu
from jax.experimental.pallas import tpu_sc as plsc

VOCAB = 1000000
EMBED_DIM = 64
BATCH = 16384
HIST = 50
NUM_IDX = BATCH * HIST

NUM_WORKERS = 32
B_PER_W = NUM_IDX // NUM_WORKERS

HALF_VOCAB = VOCAB // 2
TW = 6400
TGRID = (VOCAB + 2 * TW - 1) // (2 * TW)
EDGE_START = (TGRID - 1) * 2 * TW
EDGE_HALF = (VOCAB - EDGE_START) // 2

CHUNK = 256
WINDOWS_PER_H = BATCH // CHUNK
NBUF = 4
N_CHUNKS = B_PER_W // CHUNK
N_GROUPS = N_CHUNKS // NBUF
GATHER_W = 128


def _tc_pack(t_transposed):
    def body(x_ref, o_ref):
        i = pl.program_id(0)

        @pl.when(i < TGRID - 1)
        def _():
            o_ref[:, 0:EMBED_DIM] = x_ref[:, 0:TW].T
            o_ref[:, EMBED_DIM:] = x_ref[:, TW:].T

        @pl.when(i == TGRID - 1)
        def _():
            o_ref[0:EDGE_HALF, 0:EMBED_DIM] = x_ref[:, 0:EDGE_HALF].T
            o_ref[0:EDGE_HALF, EMBED_DIM:] = x_ref[
                :, EDGE_HALF : 2 * EDGE_HALF
            ].T

    return pl.pallas_call(
        body,
        grid=(TGRID,),
        in_specs=[pl.BlockSpec((EMBED_DIM, 2 * TW), lambda i: (0, i))],
        out_specs=pl.BlockSpec((TW, 2 * EMBED_DIM), lambda i: (i, 0)),
        out_shape=jax.ShapeDtypeStruct((HALF_VOCAB, 2 * EMBED_DIM), jnp.float32),
        compiler_params=pltpu.CompilerParams(
            dimension_semantics=("parallel",),
        ),
    )(t_transposed)


def _sc_gather(x_flat, table):
    mesh = plsc.VectorSubcoreMesh(core_axis_name="c", subcore_axis_name="s")

    @pl.kernel(
        out_type=jax.ShapeDtypeStruct((HIST, BATCH, EMBED_DIM), jnp.float32),
        mesh=mesh,
        compiler_params=pltpu.CompilerParams(use_tc_tiling_on_sc=False),
        scratch_types=(
            [
                pltpu.VMEM((B_PER_W,), jnp.int32),
                pltpu.VMEM((NBUF, CHUNK, EMBED_DIM), jnp.float32),
            ]
            + [pltpu.SemaphoreType.DMA] * (2 * NBUF)
        ),
    )
    def gather_kernel(table_hbm, idx_hbm, out_hbm, idx_all, rows_v, *sems):
        gsem = sems[:NBUF]
        wsem = sems[NBUF:]
        wid = lax.axis_index("s") * 2 + lax.axis_index("c")
        base = wid * B_PER_W
        tbase = wid * N_CHUNKS
        pltpu.sync_copy(idx_hbm.at[pl.ds(base, B_PER_W)], idx_all)

        def enq_gather(c, b):
            for j in range(CHUNK // GATHER_W):
                pltpu.async_copy(
                    table_hbm.at[
                        idx_all.at[pl.ds(c * CHUNK + j * GATHER_W, GATHER_W)]
                    ],
                    rows_v.at[b, pl.ds(j * GATHER_W, GATHER_W)],
                    gsem[b],
                )

        def wait_gather(c, b):
            for j in range(CHUNK // GATHER_W):
                pltpu.make_async_copy(
                    table_hbm.at[
                        idx_all.at[pl.ds(c * CHUNK + j * GATHER_W, GATHER_W)]
                    ],
                    rows_v.at[b, pl.ds(j * GATHER_W, GATHER_W)],
                    gsem[b],
                ).wait()

        def _dst(c):
            t = tbase + c
            h = t // WINDOWS_PER_H
            b0 = (t - h * WINDOWS_PER_H) * CHUNK
            return out_hbm.at[h, pl.ds(b0, CHUNK)]

        def enq_write(c, b):
            pltpu.async_copy(rows_v.at[b], _dst(c), wsem[b])

        def wait_write(c, b):
            pltpu.make_async_copy(rows_v.at[b], _dst(c), wsem[b]).wait()

        for b in range(NBUF):
            enq_gather(b, b)
        for b in range(NBUF):
            wait_gather(b, b)
            enq_write(b, b)

        @pl.loop(1, N_GROUPS)
        def _(g):
            c0 = g * NBUF
            for b in range(NBUF):
                wait_write(c0 - NBUF + b, b)
                enq_gather(c0 + b, b)
            for b in range(NBUF):
                wait_gather(c0 + b, b)
                enq_write(c0 + b, b)

        for b in range(NBUF):
            wait_write(N_CHUNKS - NBUF + b, b)

    return gather_kernel(table, x_flat)


HB = BATCH // 2
IDX_ROWS = NUM_IDX // 128
RPP = BATCH // 128


def _sc_idx_interleave(x128):
    mesh = plsc.VectorSubcoreMesh(core_axis_name="c", subcore_axis_name="s")

    @pl.kernel(
        out_type=jax.ShapeDtypeStruct((IDX_ROWS, 128), jnp.int32),
        mesh=mesh,
        compiler_params=pltpu.CompilerParams(needs_layout_passes=False),
        scratch_types=[
            pltpu.VMEM((RPP, 128), jnp.int32),
            pltpu.VMEM((RPP, 128), jnp.int32),
        ],
    )
    def k(x_hbm, o_hbm, plane_v, out_v):
        wid = lax.axis_index("s") * 2 + lax.axis_index("c")
        iota = lax.iota(jnp.int32, 16)
        c0 = (iota // 2) + (iota % 2) * HB

        def do_plane(h):
            pltpu.sync_copy(x_hbm.at[pl.ds(h * RPP, RPP)], plane_v)

            @pl.loop(0, RPP)
            def _(r):
                for q in range(8):
                    cols = c0 + (r * 64 + 8 * q)
                    vals = plsc.load_gather(
                        plane_v, [cols >> 7, cols & 127]
                    )
                    out_v[r, pl.ds(16 * q, 16)] = vals

            pltpu.sync_copy(out_v, o_hbm.at[pl.ds(h * RPP, RPP)])

        do_plane(wid)

        @pl.when(wid + 32 < HIST)
        def _():
            do_plane(wid + 32)

    return k(x128)


def _tc_merge(flat128):
    def body(x_ref, o_ref):
        o_ref[0, :, 0:HB] = x_ref[:, 0:EMBED_DIM].T
        o_ref[0, :, HB:] = x_ref[:, EMBED_DIM:].T

    return pl.pallas_call(
        body,
        grid=(HIST,),
        in_specs=[pl.BlockSpec((HB, 2 * EMBED_DIM), lambda h: (h, 0))],
        out_specs=pl.BlockSpec((1, EMBED_DIM, BATCH), lambda h: (h, 0, 0)),
        out_shape=jax.ShapeDtypeStruct((HIST, EMBED_DIM, BATCH), jnp.float32),
        compiler_params=pltpu.CompilerParams(
            dimension_semantics=("parallel",),
        ),
    )(flat128)


@jax.jit
def kernel(x, table):
    xi = x.astype(jnp.int32)
    blk = xi // (2 * TW)
    j = xi - blk * (2 * TW)
    half = j // TW
    xm = blk * (2 * TW) + 2 * (j - half * TW) + half
    je = xi - EDGE_START
    he = je // EDGE_HALF
    xm_edge = EDGE_START + 2 * (je - he * EDGE_HALF) + he
    xm = jnp.where(xi < EDGE_START, xm, xm_edge)

    x128 = xm.T.reshape(IDX_ROWS, 128)
    x_lin = _sc_idx_interleave(x128).reshape(NUM_IDX)

    table_lin = _tc_pack(table.T).reshape(VOCAB, EMBED_DIM)
    hmajor = _sc_gather(x_lin, table_lin)
    out = _tc_merge(hmajor.reshape(NUM_IDX // 2, 2 * EMBED_DIM))
    return out.transpose(2, 0, 1)


# speedup vs baseline: 1.6954x; 1.0390x over previous
import jax
import jax.numpy as jnp
from jax import lax
from jax.experimental import pallas as pl
from jax.experimental.pallas import tpu as pltpu
from jax.experimental.pallas import tpu_sc as plsc

VOCAB = 1000000
EMBED_DIM = 64
BATCH = 16384
HIST = 50
NUM_IDX = BATCH * HIST

NUM_WORKERS = 32
B_PER_W = NUM_IDX // NUM_WORKERS

HALF_VOCAB = VOCAB // 2
TW = 12800
TGRID = (VOCAB + 2 * TW - 1) // (2 * TW)
EDGE_START = (TGRID - 1) * 2 * TW
EDGE_HALF = (VOCAB - EDGE_START) // 2

CHUNK = 256
WINDOWS_PER_H = BATCH // CHUNK
NBUF = 4
N_CHUNKS = B_PER_W // CHUNK
N_GROUPS = N_CHUNKS // NBUF
GATHER_W = 128


def _tc_pack(t_transposed):
    def body(x_ref, o_ref):
        i = pl.program_id(0)

        @pl.when(i < TGRID - 1)
        def _():
            o_ref[:, 0:EMBED_DIM] = x_ref[:, 0:TW].T
            o_ref[:, EMBED_DIM:] = x_ref[:, TW:].T

        @pl.when(i == TGRID - 1)
        def _():
            o_ref[0:EDGE_HALF, 0:EMBED_DIM] = x_ref[:, 0:EDGE_HALF].T
            o_ref[0:EDGE_HALF, EMBED_DIM:] = x_ref[
                :, EDGE_HALF : 2 * EDGE_HALF
            ].T

    return pl.pallas_call(
        body,
        grid=(TGRID,),
        in_specs=[pl.BlockSpec((EMBED_DIM, 2 * TW), lambda i: (0, i))],
        out_specs=pl.BlockSpec((TW, 2 * EMBED_DIM), lambda i: (i, 0)),
        out_shape=jax.ShapeDtypeStruct((HALF_VOCAB, 2 * EMBED_DIM), jnp.float32),
        compiler_params=pltpu.CompilerParams(
            dimension_semantics=("parallel",),
        ),
    )(t_transposed)


def _sc_gather(x_flat, table):
    mesh = plsc.VectorSubcoreMesh(core_axis_name="c", subcore_axis_name="s")

    @pl.kernel(
        out_type=jax.ShapeDtypeStruct((HIST, BATCH, EMBED_DIM), jnp.float32),
        mesh=mesh,
        compiler_params=pltpu.CompilerParams(use_tc_tiling_on_sc=False),
        scratch_types=(
            [
                pltpu.VMEM((B_PER_W,), jnp.int32),
                pltpu.VMEM((NBUF, CHUNK, EMBED_DIM), jnp.float32),
            ]
            + [pltpu.SemaphoreType.DMA] * (2 * NBUF)
        ),
    )
    def gather_kernel(table_hbm, idx_hbm, out_hbm, idx_all, rows_v, *sems):
        gsem = sems[:NBUF]
        wsem = sems[NBUF:]
        wid = lax.axis_index("s") * 2 + lax.axis_index("c")
        base = wid * B_PER_W
        tbase = wid * N_CHUNKS
        pltpu.sync_copy(idx_hbm.at[pl.ds(base, B_PER_W)], idx_all)

        def enq_gather(c, b):
            for j in range(CHUNK // GATHER_W):
                pltpu.async_copy(
                    table_hbm.at[
                        idx_all.at[pl.ds(c * CHUNK + j * GATHER_W, GATHER_W)]
                    ],
                    rows_v.at[b, pl.ds(j * GATHER_W, GATHER_W)],
                    gsem[b],
                )

        def wait_gather(c, b):
            for j in range(CHUNK // GATHER_W):
                pltpu.make_async_copy(
                    table_hbm.at[
                        idx_all.at[pl.ds(c * CHUNK + j * GATHER_W, GATHER_W)]
                    ],
                    rows_v.at[b, pl.ds(j * GATHER_W, GATHER_W)],
                    gsem[b],
                ).wait()

        def _dst(c):
            t = tbase + c
            h = t // WINDOWS_PER_H
            b0 = (t - h * WINDOWS_PER_H) * CHUNK
            return out_hbm.at[h, pl.ds(b0, CHUNK)]

        def enq_write(c, b):
            pltpu.async_copy(rows_v.at[b], _dst(c), wsem[b])

        def wait_write(c, b):
            pltpu.make_async_copy(rows_v.at[b], _dst(c), wsem[b]).wait()

        for b in range(NBUF):
            enq_gather(b, b)
        for b in range(NBUF):
            wait_gather(b, b)
            enq_write(b, b)

        @pl.loop(1, N_GROUPS)
        def _(g):
            c0 = g * NBUF
            for b in range(NBUF):
                wait_write(c0 - NBUF + b, b)
                enq_gather(c0 + b, b)
            for b in range(NBUF):
                wait_gather(c0 + b, b)
                enq_write(c0 + b, b)

        for b in range(NBUF):
            wait_write(N_CHUNKS - NBUF + b, b)

    return gather_kernel(table, x_flat)


HB = BATCH // 2
IDX_ROWS = NUM_IDX // 128
RPP = BATCH // 128


def _sc_idx_interleave(x128):
    mesh = plsc.VectorSubcoreMesh(core_axis_name="c", subcore_axis_name="s")

    @pl.kernel(
        out_type=jax.ShapeDtypeStruct((IDX_ROWS, 128), jnp.int32),
        mesh=mesh,
        compiler_params=pltpu.CompilerParams(needs_layout_passes=False),
        scratch_types=[
            pltpu.VMEM((RPP, 128), jnp.int32),
            pltpu.VMEM((RPP, 128), jnp.int32),
        ],
    )
    def k(x_hbm, o_hbm, plane_v, out_v):
        wid = lax.axis_index("s") * 2 + lax.axis_index("c")
        iota = lax.iota(jnp.int32, 16)
        c0 = (iota // 2) + (iota % 2) * HB

        def do_plane(h):
            pltpu.sync_copy(x_hbm.at[pl.ds(h * RPP, RPP)], plane_v)

            @pl.loop(0, RPP)
            def _(r):
                for q in range(8):
                    cols = c0 + (r * 64 + 8 * q)
                    vals = plsc.load_gather(
                        plane_v, [cols >> 7, cols & 127]
                    )
                    out_v[r, pl.ds(16 * q, 16)] = vals

            pltpu.sync_copy(out_v, o_hbm.at[pl.ds(h * RPP, RPP)])

        do_plane(wid)

        @pl.when(wid + 32 < HIST)
        def _():
            do_plane(wid + 32)

    return k(x128)


def _tc_merge(flat128):
    def body(x_ref, o_ref):
        o_ref[0, :, 0:HB] = x_ref[:, 0:EMBED_DIM].T
        o_ref[0, :, HB:] = x_ref[:, EMBED_DIM:].T

    return pl.pallas_call(
        body,
        grid=(HIST,),
        in_specs=[pl.BlockSpec((HB, 2 * EMBED_DIM), lambda h: (h, 0))],
        out_specs=pl.BlockSpec((1, EMBED_DIM, BATCH), lambda h: (h, 0, 0)),
        out_shape=jax.ShapeDtypeStruct((HIST, EMBED_DIM, BATCH), jnp.float32),
        compiler_params=pltpu.CompilerParams(
            dimension_semantics=("parallel",),
        ),
    )(flat128)


@jax.jit
def kernel(x, table):
    xi = x.astype(jnp.int32)
    blk = xi // (2 * TW)
    j = xi - blk * (2 * TW)
    half = j // TW
    xm = blk * (2 * TW) + 2 * (j - half * TW) + half
    je = xi - EDGE_START
    he = je // EDGE_HALF
    xm_edge = EDGE_START + 2 * (je - he * EDGE_HALF) + he
    xm = jnp.where(xi < EDGE_START, xm, xm_edge)

    x128 = xm.T.reshape(IDX_ROWS, 128)
    x_lin = _sc_idx_interleave(x128).reshape(NUM_IDX)

    table_lin = _tc_pack(table.T).reshape(VOCAB, EMBED_DIM)
    hmajor = _sc_gather(x_lin, table_lin)
    out = _tc_merge(hmajor.reshape(NUM_IDX // 2, 2 * EMBED_DIM))
    return out.transpose(2, 0, 1)
